# baseline (speedup 1.0000x reference)
import jax
import jax.numpy as jnp
from jax import lax
from jax.experimental import pallas as pl
from jax.experimental.pallas import tpu as pltpu
from jax.experimental.pallas import tpu_sc as plsc

_NUM_TYPES = 16
_HIDDEN = 512
_EMBED = 768
_B = 16384

_NC = 2
_NS = 16
_NW = _NC * _NS
_BPW = _B // _NW
_CHUNK = 64
_NCHUNK = _BPW // _CHUNK


def _mlp_body(table_ref, w1_ref, b1_ref, w2_ref, b2_ref, out_ref):
    h = jnp.dot(table_ref[...], w1_ref[...], preferred_element_type=jnp.float32)
    h = h + b1_ref[...]
    h = h * jax.nn.sigmoid(h)
    out_ref[...] = (
        jnp.dot(h, w2_ref[...], preferred_element_type=jnp.float32) + b2_ref[...]
    )


def _mlp_table(table, W1, b1, W2, b2):
    return pl.pallas_call(
        _mlp_body,
        out_shape=jax.ShapeDtypeStruct((_NUM_TYPES, _EMBED), jnp.float32),
    )(table, W1, b1.reshape(1, _HIDDEN), W2, b2.reshape(1, _EMBED))


def _gather_body(tab_hbm, idx_hbm, out_hbm, idx_v, rows0, rows1, g0, g1, s0, s1):
    wid = lax.axis_index("s") * _NC + lax.axis_index("c")
    base = wid * _BPW
    pltpu.sync_copy(idx_hbm.at[pl.ds(base, _BPW)], idx_v)

    rows = (rows0, rows1)
    gsem = (g0, g1)
    ssem = (s0, s1)

    def src(c):
        return tab_hbm.at[idx_v.at[pl.ds(c * _CHUNK, _CHUNK)]]

    def dst(c):
        return out_hbm.at[pl.ds(base + c * _CHUNK, _CHUNK)]

    def gstart(c):
        pltpu.async_copy(src(c), rows[c % 2], gsem[c % 2])

    def gwait(c):
        pltpu.make_async_copy(src(c), rows[c % 2], gsem[c % 2]).wait()

    def sstart(c):
        pltpu.async_copy(rows[c % 2], dst(c), ssem[c % 2])

    def swait(c):
        pltpu.make_async_copy(rows[c % 2], dst(c), ssem[c % 2]).wait()

    gstart(0)
    gstart(1)
    for c in range(_NCHUNK):
        gwait(c)
        sstart(c)
        if c + 2 < _NCHUNK:
            swait(c)
            gstart(c + 2)
    swait(_NCHUNK - 2)
    swait(_NCHUNK - 1)


def _gather(out_table, tok):
    mesh = plsc.VectorSubcoreMesh(
        core_axis_name="c", subcore_axis_name="s", num_cores=_NC
    )
    run = pl.kernel(
        _gather_body,
        out_type=jax.ShapeDtypeStruct((_B, _EMBED), jnp.float32),
        mesh=mesh,
        scratch_types=[
            pltpu.VMEM((_BPW,), jnp.int32),
            pltpu.VMEM((_CHUNK, _EMBED), jnp.float32),
            pltpu.VMEM((_CHUNK, _EMBED), jnp.float32),
            pltpu.SemaphoreType.DMA,
            pltpu.SemaphoreType.DMA,
            pltpu.SemaphoreType.DMA,
            pltpu.SemaphoreType.DMA,
        ],
    )
    return run(out_table, tok)


def kernel(goal_tokens, table, W1, b1, W2, b2):
    tok = goal_tokens.astype(jnp.int32)
    out_table = _mlp_table(table, W1, b1, W2, b2)
    return _gather(out_table, tok)
